# Initial kernel scaffold; baseline (speedup 1.0000x reference)
import jax
import jax.numpy as jnp
from jax.experimental import pallas as pl


def kernel(x, edge_index, enc_W1, enc_b1, enc_g, enc_beta, enc_W2, enc_b2, proj_W, proj_b, Ws, bs, Wn, bn, g, beta, head_W1, head_b1, head_W2, head_b2):
    raise NotImplementedError("write your pallas kernel here")


# speedup vs baseline: 3.0827x; 3.0827x over previous
import functools

import jax
import jax.numpy as jnp
from jax import lax
from jax.experimental import pallas as pl
from jax.experimental.pallas import tpu as pltpu
from jax.experimental.pallas import tpu_sc as plsc

N = 10000
D = 128
E = 320000
L = 4
EPS = 1e-5

NC = 2
NS = 16
NW = NC * NS
K = 128
CHUNKS = -(-E // (NW * K))
WE = CHUNKS * K
EP = NW * WE
RPT = 632
N_ACC = NS * RPT
DEG_W = 16

_MESH = plsc.VectorSubcoreMesh(core_axis_name="c", subcore_axis_name="s")

f32 = jnp.float32


def _seg_body(z_hbm, row_hbm, col_hbm, zD_hbm, out_hbm,
              acc_sh, colbuf, rowbuf, rows_v, sem):
    cid = lax.axis_index("c")
    sid = lax.axis_index("s")
    wid = sid * NC + cid
    r0 = sid * RPT

    pltpu.sync_copy(zD_hbm.at[pl.ds(r0, RPT)], acc_sh.at[pl.ds(r0, RPT)])
    plsc.subcore_barrier()

    def step(j, c):
        base = wid * WE + j * K
        pltpu.sync_copy(col_hbm.at[pl.ds(base, K)], colbuf)
        pltpu.sync_copy(row_hbm.at[pl.ds(base, K)], rowbuf)
        pltpu.async_copy(z_hbm.at[colbuf], rows_v, sem).wait()
        pltpu.sync_copy(rows_v, acc_sh.at[rowbuf], add=True)
        return c
    lax.fori_loop(0, CHUNKS, step, 0)

    plsc.subcore_barrier()
    pltpu.sync_copy(acc_sh.at[pl.ds(r0, RPT)],
                    out_hbm.at[pl.ds(cid * N_ACC + r0, RPT)])


_seg_call = pl.kernel(
    _seg_body,
    out_type=jax.ShapeDtypeStruct((NC * N_ACC, D), f32),
    mesh=_MESH,
    scratch_types=[
        pltpu.VMEM_SHARED((N_ACC, D), f32),
        pltpu.VMEM((K,), jnp.int32),
        pltpu.VMEM((K,), jnp.int32),
        pltpu.VMEM((K, D), f32),
        pltpu.SemaphoreType.DMA,
    ],
)


RB = 2000
GRID = N // RB


def _full(shape):
    nd = len(shape)
    return pl.BlockSpec(shape, lambda r: (0,) * nd)


def _rows(width=D):
    return pl.BlockSpec((RB, width), lambda r: (r, 0))


def _dot(a, b):
    return jnp.dot(a, b, preferred_element_type=f32)


def _pre_body(x_ref, W1, b1, s1, be, W2, b2, Wp, bp, Wn0, Ws0, bs0,
              h_ref, z_ref, lin_ref):
    t = _dot(x_ref[...], W1[...]) + b1[...]
    t = jnp.maximum(t * s1[...] + be[...], 0.0)
    t = _dot(t, W2[...]) + b2[...]
    h = _dot(t, Wp[...]) + bp[...]
    h_ref[...] = h
    z_ref[...] = _dot(h, Wn0[...])
    lin_ref[...] = _dot(h, Ws0[...]) + bs0[...]


_pre_call = pl.pallas_call(
    _pre_body,
    grid=(GRID,),
    in_specs=[_rows(), _full((D, D)), _full((1, D)), _full((1, D)),
              _full((1, D)), _full((D, D)), _full((1, D)), _full((D, D)),
              _full((1, D)), _full((D, D)), _full((D, D)), _full((1, D))],
    out_specs=[_rows(), _rows(), _rows()],
    out_shape=[jax.ShapeDtypeStruct((N, D), f32)] * 3,
)


def _mid_body(first, *refs):
    if first:
        (h_ref, part_ref, degp_ref, lin_ref, bnb, scl, bet, Wnn, Wsn, bsn,
         hn_ref, zn_ref, linn_ref, inv_ref) = refs
        degs = degp_ref[...]
        deg = jnp.maximum((degs[0] + degs[1])[:, 0:1], 1.0)
        inv = 1.0 / deg
        inv_ref[...] = inv
    else:
        (h_ref, part_ref, inv_ref, lin_ref, bnb, scl, bet, Wnn, Wsn, bsn,
         hn_ref, zn_ref, linn_ref) = refs
        inv = inv_ref[...]
    part = part_ref[...]
    s = part[0] + part[1]
    o = lin_ref[...] + s * inv + bnb[...]
    o = jnp.maximum(o * scl[...] + bet[...], 0.0)
    hn = h_ref[...] + o
    hn_ref[...] = hn
    zn_ref[...] = _dot(hn, Wnn[...])
    linn_ref[...] = _dot(hn, Wsn[...]) + bsn[...]


_part_spec = pl.BlockSpec((NC, RB, D), lambda r: (0, r, 0))
_degp_spec = pl.BlockSpec((NC, RB, D), lambda r: (0, r, 0))
_inv_spec = pl.BlockSpec((RB, 1), lambda r: (r, 0))

_mid0_call = pl.pallas_call(
    functools.partial(_mid_body, True),
    grid=(GRID,),
    in_specs=[_rows(), _part_spec, _degp_spec, _rows(), _full((1, D)),
              _full((1, D)), _full((1, D)), _full((D, D)), _full((D, D)),
              _full((1, D))],
    out_specs=[_rows(), _rows(), _rows(), _inv_spec],
    out_shape=[jax.ShapeDtypeStruct((N, D), f32)] * 3
    + [jax.ShapeDtypeStruct((N, 1), f32)],
)

_mid_call = pl.pallas_call(
    functools.partial(_mid_body, False),
    grid=(GRID,),
    in_specs=[_rows(), _part_spec, _inv_spec, _rows(), _full((1, D)),
              _full((1, D)), _full((1, D)), _full((D, D)), _full((D, D)),
              _full((1, D))],
    out_specs=[_rows(), _rows(), _rows()],
    out_shape=[jax.ShapeDtypeStruct((N, D), f32)] * 3,
)


def _fin_body(h_ref, part_ref, inv_ref, lin_ref, bnb, scl, bet,
              hW1, hb1, hW2, hb2, out_ref):
    part = part_ref[...]
    s = part[0] + part[1]
    o = lin_ref[...] + s * inv_ref[...] + bnb[...]
    o = jnp.maximum(o * scl[...] + bet[...], 0.0)
    hf = h_ref[...] + o
    y = jnp.maximum(_dot(hf, hW1[...]) + hb1[...], 0.0)
    out_ref[...] = _dot(y, hW2[...]) + hb2[...]


_fin_call = pl.pallas_call(
    _fin_body,
    grid=(GRID,),
    in_specs=[_rows(), _part_spec, _inv_spec, _rows(), _full((1, D)),
              _full((1, D)), _full((1, D)), _full((D, D // 2)),
              _full((1, D // 2)), _full((D // 2, 1)), _full((1, 1))],
    out_specs=[_inv_spec],
    out_shape=[jax.ShapeDtypeStruct((N, 1), f32)],
)


def kernel(x, edge_index, enc_W1, enc_b1, enc_g, enc_beta, enc_W2, enc_b2,
           proj_W, proj_b, Ws, bs, Wn, bnb, g, beta,
           head_W1, head_b1, head_W2, head_b2):
    row = edge_index[0]
    col = edge_index[1]
    pad = EP - E
    rowp = jnp.concatenate([row, jnp.full((pad,), N, jnp.int32)])
    colp = jnp.concatenate([col, jnp.zeros((pad,), jnp.int32)])
    zerosD = jnp.zeros((N_ACC, D), f32)
    onesND = jnp.ones((N, D), f32)

    r1 = lambda v: v.reshape(1, -1)
    bn_scale = 1.0 / jnp.sqrt(1.0 + EPS)
    s_enc = r1(enc_g * bn_scale)
    scl = g * bn_scale

    h, z, lin = _pre_call(x, enc_W1, r1(enc_b1), s_enc, r1(enc_beta),
                          enc_W2, r1(enc_b2), proj_W, r1(proj_b),
                          Wn[0], Ws[0], r1(bs[0]))

    part = _seg_call(z, rowp, colp, zerosD).reshape(NC, N_ACC, D)
    degp = _seg_call(onesND, rowp, colp, zerosD).reshape(NC, N_ACC, D)
    h, z, lin, inv = _mid0_call(h, part, degp, lin, r1(bnb[0]), r1(scl[0]),
                                r1(beta[0]), Wn[1], Ws[1], r1(bs[1]))
    for i in (1, 2):
        part = _seg_call(z, rowp, colp, zerosD).reshape(NC, N_ACC, D)
        h, z, lin = _mid_call(h, part, inv, lin, r1(bnb[i]), r1(scl[i]),
                              r1(beta[i]), Wn[i + 1], Ws[i + 1],
                              r1(bs[i + 1]))
    part = _seg_call(z, rowp, colp, zerosD).reshape(NC, N_ACC, D)
    (out,) = _fin_call(h, part, inv, lin, r1(bnb[3]), r1(scl[3]), r1(beta[3]),
                       head_W1, r1(head_b1), head_W2, head_b2.reshape(1, 1))
    return out[:, 0]
